# Initial kernel scaffold; baseline (speedup 1.0000x reference)
import jax
import jax.numpy as jnp
from jax.experimental import pallas as pl


def kernel(x, w1, b1, w2, b2):
    raise NotImplementedError("write your pallas kernel here")


# speedup vs baseline: 426.2970x; 426.2970x over previous
import jax
import jax.numpy as jnp
from jax.experimental import pallas as pl
from jax.experimental.pallas import tpu as pltpu

_B = 16384
_R = 128
_L = 64
_PHASE1 = 10
_STEPS = {-4: 1342177, -3: 671089, -2: 335544, -1: 167772,
          0: 83886, 1: 41943, 2: 20972, 3: 10486}


def _path_explosion_body(xt_ref, w1_ref, b1_ref, w2_ref, b2_ref, out_ref):
    f32 = jnp.float32
    x0 = xt_ref[0]
    x1 = xt_ref[1]

    acc = jnp.zeros((_R, _R), f32)
    for j in range(_L):
        h = jnp.maximum(x0 * w1_ref[0, j] + x1 * w1_ref[1, j] + b1_ref[j], 0.0)
        acc = acc + h * w2_ref[j, 0]
    z = acc + b2_ref[0]
    bound = 1.0 / (1.0 + jnp.exp(-z))
    bnd = bound[None, :, :]

    xv = xt_ref[...]
    for _ in range(_PHASE1):
        xv = xv + f32(0.01)
        bi = xv <= bnd
        b2i = xv <= bnd - f32(0.001)
        xv = jnp.where(jnp.logical_and(bi, b2i), xv * 2.0, xv)
        xv = jnp.where(jnp.logical_and(bi, jnp.logical_not(b2i)), xv * 3.0, xv)

    k = jnp.full(xv.shape, _PHASE1, jnp.int32)
    for e in range(-4, 4):
        lo = f32(2.0 ** e)
        if e == 3:
            active = jnp.logical_and(xv >= lo, xv <= f32(10.0))
            m_target = 10 * (1 << 20) + 1
        else:
            active = jnp.logical_and(xv >= lo, xv < f32(2.0 ** (e + 1)))
            m_target = 1 << 24
        s = _STEPS[e]
        m = (xv * f32(2.0 ** (23 - e))).astype(jnp.int32)
        diff = jnp.maximum(m_target - 1 - m, 0)
        q = (diff.astype(f32) * f32(1.0 / s)).astype(jnp.int32)
        q = jnp.where(q * s > diff, q - 1, q)
        q = jnp.where((q + 1) * s <= diff, q + 1, q)
        xj = (m + q * s).astype(f32) * f32(2.0 ** (e - 23))
        xn = xj + f32(0.01)
        xv = jnp.where(active, xn, xv)
        k = jnp.where(active, k + q + 1, k)

    out_ref[...] = k.astype(f32)


def kernel(x, w1, b1, w2, b2):
    xt = x.T.reshape(2, _R, _R)
    out = pl.pallas_call(
        _path_explosion_body,
        out_shape=jax.ShapeDtypeStruct((2, _R, _R), jnp.float32),
        in_specs=[
            pl.BlockSpec(memory_space=pltpu.VMEM),
            pl.BlockSpec(memory_space=pltpu.SMEM),
            pl.BlockSpec(memory_space=pltpu.SMEM),
            pl.BlockSpec(memory_space=pltpu.SMEM),
            pl.BlockSpec(memory_space=pltpu.SMEM),
        ],
        out_specs=pl.BlockSpec(memory_space=pltpu.VMEM),
    )(xt, w1, b1, w2, b2)
    return out.reshape(2, _B).T
